# baseline (speedup 1.0000x reference)
import functools

import jax
import jax.numpy as jnp
from jax.experimental import pallas as pl
from jax.experimental.pallas import tpu as pltpu

_C0 = 0.28209479177387814
_C1 = 0.4886025119029199
_K = 16


def _nbody_block(t_ref, l_ref, wsrc_ref, wdst_ref, w5_ref, bm_ref,
                 woagg_ref, wox_ref, wv_ref, bo_ref, out_ref,
                 *, bsz, n, wave):
    f32 = jnp.float32
    tblk = t_ref[...]
    massT = tblk[:, 0:1, :]
    locT = tblk[:, 1:4, :]
    velT = tblk[:, 4:7, :]

    wsum = jnp.sum(massT, axis=2, keepdims=True)
    comT = jnp.sum(massT / wsum * locT, axis=2, keepdims=True)
    posT = locT - comT
    vabsT = jnp.sqrt(jnp.sum(velT * velT, axis=1, keepdims=True))

    jsub = jax.lax.broadcasted_iota(jnp.int32, (n, n), 0)
    diag = jsub == jax.lax.broadcasted_iota(jnp.int32, (n, n), 1)
    ones_row = jnp.ones((1, n), dtype=f32)

    for w0 in range(0, bsz, wave):
        _nbody_wave(range(w0, w0 + wave), massT, velT, vabsT, comT, posT,
                    l_ref, wsrc_ref, wdst_ref, w5_ref, bm_ref, woagg_ref,
                    wox_ref, wv_ref, bo_ref, out_ref, jsub, diag, ones_row, n)


def _nbody_wave(sysr, massT, velT, vabsT, comT, posT, l_ref, wsrc_ref,
                wdst_ref, w5_ref, bm_ref, woagg_ref, wox_ref, wv_ref,
                bo_ref, out_ref, jsub, diag, ones_row, n):
    f32 = jnp.float32
    featT, pmT, botT, pw, d2T = {}, {}, {}, {}, {}
    for b in sysr:
        ft = jnp.concatenate(
            [posT[b], velT[b], vabsT[b], massT[b]], axis=0)
        featT[b] = ft
        pmT[b] = jnp.concatenate([posT[b], massT[b]], axis=0)
        botT[b] = jnp.concatenate([ft[0:7], ones_row], axis=0)
        p_tab = jnp.dot(wsrc_ref[...], ft[0:7], preferred_element_type=f32)
        pw[b] = jnp.concatenate(
            [p_tab, w5_ref[...], wdst_ref[...], bm_ref[...]], axis=1)

        pos_s = l_ref[b] - jnp.transpose(comT[b], (1, 0))
        acc = None
        for c in range(3):
            dc = pos_s[:, c:c + 1] - posT[b][c:c + 1, :]
            acc = dc * dc if acc is None else acc + dc * dc
        acc = jnp.where(diag, acc + 1e10, acc)
        ikey = (jax.lax.bitcast_convert_type(acc, jnp.int32)
                & jnp.int32(-128)) | jsub
        d2T[b] = jax.lax.bitcast_convert_type(ikey, f32)

    accm = {b: None for b in sysr}
    accu = {b: None for b in sysr}
    for k in range(_K):
        for b in sysr:
            mn = jnp.min(d2T[b], axis=0, keepdims=True)
            eqT = d2T[b] == mn
            ohf = eqT.astype(f32)
            d2T[b] = d2T[b] + ohf * f32(3.0e38)

            d2e = jax.lax.bitcast_convert_type(
                jax.lax.bitcast_convert_type(mn, jnp.int32)
                & jnp.int32(-128), f32)
            rsq = jax.lax.rsqrt(jnp.maximum(d2e, 1e-16))
            dist = d2e * rsq
            g4 = jnp.dot(pmT[b], ohf, preferred_element_type=f32)
            relT = g4[0:3] - posT[b]
            u = relT * rsq
            mprod = g4[3:4] * massT[b]
            rows = jnp.concatenate([ohf, u, dist, mprod, botT[b]], axis=0)
            pre = jnp.dot(pw[b], rows, preferred_element_type=f32)
            mk = jnp.maximum(pre, 0.0)
            accm[b] = mk if accm[b] is None else accm[b] + mk
            accu[b] = u if accu[b] is None else accu[b] + u

    for b in sysr:
        uvT = featT[b][3:6] / jnp.maximum(featT[b][6:7], 1e-8)
        wvecT = accu[b] * (1.0 / _K) + uvT
        shiftT = (jnp.dot(wox_ref[...], featT[b][0:7],
                          preferred_element_type=f32)
                  + jnp.dot(woagg_ref[...], accm[b],
                            preferred_element_type=f32)
                  + jnp.dot(wv_ref[...], wvecT, preferred_element_type=f32)
                  + bo_ref[...])
        out_ref[b] = posT[b] + shiftT + comT[b]


def kernel(inputs, W_m, b_m, W_o, b_o):
    batchsize, n, _ = inputs.shape
    bsz = 50 if batchsize % 50 == 0 else 1
    f32 = jnp.float32

    tin = inputs.transpose(0, 2, 1)
    lin = inputs[:, :, 1:4]

    wsrct = W_m[0:7].T
    wdstt = W_m[7:14].T
    w5t = jnp.concatenate(
        [_C1 * W_m[jnp.array([17, 15, 16])], W_m[18:20]], axis=0).T
    bm2 = (b_m + _C0 * W_m[14]).reshape(-1, 1)
    woxt = W_o[0:7].T
    woaggt = W_o[7:135].T
    wvt = (_C1 * W_o[jnp.array([138, 136, 137])]).T
    bo2 = (b_o + 2.0 * _C0 * W_o[135]).reshape(-1, 1)

    grid = (batchsize // bsz,)
    body = functools.partial(_nbody_block, bsz=bsz, n=n, wave=bsz)
    preds_t = pl.pallas_call(
        body,
        grid=grid,
        in_specs=[
            pl.BlockSpec((bsz, 7, n), lambda i: (i, 0, 0)),
            pl.BlockSpec((bsz, n, 3), lambda i: (i, 0, 0)),
            pl.BlockSpec((128, 7), lambda i: (0, 0)),
            pl.BlockSpec((128, 7), lambda i: (0, 0)),
            pl.BlockSpec((128, 5), lambda i: (0, 0)),
            pl.BlockSpec((128, 1), lambda i: (0, 0)),
            pl.BlockSpec((3, 128), lambda i: (0, 0)),
            pl.BlockSpec((3, 7), lambda i: (0, 0)),
            pl.BlockSpec((3, 3), lambda i: (0, 0)),
            pl.BlockSpec((3, 1), lambda i: (0, 0)),
        ],
        out_specs=pl.BlockSpec((bsz, 3, n), lambda i: (i, 0, 0)),
        out_shape=jax.ShapeDtypeStruct((batchsize, 3, n), f32),
        compiler_params=pltpu.CompilerParams(
            dimension_semantics=("parallel",)),
    )(tin, lin, wsrct, wdstt, w5t, bm2, woaggt, woxt, wvt, bo2)
    preds = preds_t.transpose(0, 2, 1)
    return preds, jnp.zeros((batchsize,), dtype=f32)


# speedup vs baseline: 1.1329x; 1.1329x over previous
import functools

import jax
import jax.numpy as jnp
from jax.experimental import pallas as pl
from jax.experimental.pallas import tpu as pltpu

_C0 = 0.28209479177387814
_C1 = 0.4886025119029199
_K = 16


def _nbody_block(t_ref, l_ref, wsrc_ref, wdst_ref, w5_ref, bm_ref,
                 woagg_ref, wox_ref, wv_ref, bo_ref, out_ref,
                 *, bsz, n, wave):
    f32 = jnp.float32
    tblk = t_ref[...]
    massT = tblk[:, 0:1, :]
    locT = tblk[:, 1:4, :]
    velT = tblk[:, 4:7, :]

    wsum = jnp.sum(massT, axis=2, keepdims=True)
    comT = jnp.sum(massT / wsum * locT, axis=2, keepdims=True)
    posT = locT - comT
    vabsT = jnp.sqrt(jnp.sum(velT * velT, axis=1, keepdims=True))

    jsub = jax.lax.broadcasted_iota(jnp.int32, (n, n), 0)
    diag = jsub == jax.lax.broadcasted_iota(jnp.int32, (n, n), 1)
    ones_row = jnp.ones((1, n), dtype=f32)

    for w0 in range(0, bsz, wave):
        _nbody_wave(range(w0, w0 + wave), massT, velT, vabsT, comT, posT,
                    l_ref, wsrc_ref, wdst_ref, w5_ref, bm_ref, woagg_ref,
                    wox_ref, wv_ref, bo_ref, out_ref, jsub, diag, ones_row, n)


def _nbody_wave(sysr, massT, velT, vabsT, comT, posT, l_ref, wsrc_ref,
                wdst_ref, w5_ref, bm_ref, woagg_ref, wox_ref, wv_ref,
                bo_ref, out_ref, jsub, diag, ones_row, n):
    f32 = jnp.float32
    featT, pmT, botT, pw, d2T = {}, {}, {}, {}, {}
    for b in sysr:
        ft = jnp.concatenate(
            [posT[b], velT[b], vabsT[b], massT[b]], axis=0)
        featT[b] = ft
        pmT[b] = jnp.concatenate([posT[b], massT[b]], axis=0)
        botT[b] = jnp.concatenate([ft[0:7], ones_row], axis=0)
        p_tab = jnp.dot(wsrc_ref[...], ft[0:7], preferred_element_type=f32)
        pw[b] = jnp.concatenate(
            [p_tab, w5_ref[...], wdst_ref[...], bm_ref[...]], axis=1)

        pos_s = l_ref[b] - jnp.transpose(comT[b], (1, 0))
        acc = None
        for c in range(3):
            dc = pos_s[:, c:c + 1] - posT[b][c:c + 1, :]
            acc = dc * dc if acc is None else acc + dc * dc
        acc = jnp.where(diag, acc + 1e10, acc)
        ikey = (jax.lax.bitcast_convert_type(acc, jnp.int32)
                & jnp.int32(-128)) | jsub
        d2T[b] = jax.lax.bitcast_convert_type(ikey, f32)

    accm = {b: None for b in sysr}
    accu = {b: None for b in sysr}
    for k in range(_K):
        for b in sysr:
            mn = jnp.min(d2T[b], axis=0, keepdims=True)
            eqT = d2T[b] == mn
            ohf = eqT.astype(f32)
            d2T[b] = d2T[b] + ohf * f32(3.0e38)

            d2e = jax.lax.bitcast_convert_type(
                jax.lax.bitcast_convert_type(mn, jnp.int32)
                & jnp.int32(-128), f32)
            rsq = jax.lax.rsqrt(jnp.maximum(d2e, 1e-16))
            dist = d2e * rsq
            g4 = jnp.dot(pmT[b], ohf, preferred_element_type=f32)
            relT = g4[0:3] - posT[b]
            u = relT * rsq
            mprod = g4[3:4] * massT[b]
            rows = jnp.concatenate([ohf, u, dist, mprod, botT[b]], axis=0)
            pre = jnp.dot(pw[b], rows, preferred_element_type=f32)
            mk = jnp.maximum(pre, 0.0)
            accm[b] = mk if accm[b] is None else accm[b] + mk
            accu[b] = u if accu[b] is None else accu[b] + u

    for b in sysr:
        uvT = featT[b][3:6] / jnp.maximum(featT[b][6:7], 1e-8)
        wvecT = accu[b] * (1.0 / _K) + uvT
        shiftT = (jnp.dot(wox_ref[...], featT[b][0:7],
                          preferred_element_type=f32)
                  + jnp.dot(woagg_ref[...], accm[b],
                            preferred_element_type=f32)
                  + jnp.dot(wv_ref[...], wvecT, preferred_element_type=f32)
                  + bo_ref[...])
        out_ref[b] = posT[b] + shiftT + comT[b]


def kernel(inputs, W_m, b_m, W_o, b_o):
    batchsize, n, _ = inputs.shape
    bsz = 40 if batchsize % 40 == 0 else 1
    f32 = jnp.float32

    tin = inputs.transpose(0, 2, 1)
    lin = inputs[:, :, 1:4]

    wsrct = W_m[0:7].T
    wdstt = W_m[7:14].T
    w5t = jnp.concatenate(
        [_C1 * W_m[jnp.array([17, 15, 16])], W_m[18:20]], axis=0).T
    bm2 = (b_m + _C0 * W_m[14]).reshape(-1, 1)
    woxt = W_o[0:7].T
    woaggt = W_o[7:135].T
    wvt = (_C1 * W_o[jnp.array([138, 136, 137])]).T
    bo2 = (b_o + 2.0 * _C0 * W_o[135]).reshape(-1, 1)

    grid = (batchsize // bsz,)
    body = functools.partial(_nbody_block, bsz=bsz, n=n, wave=min(8, bsz))
    preds_t = pl.pallas_call(
        body,
        grid=grid,
        in_specs=[
            pl.BlockSpec((bsz, 7, n), lambda i: (i, 0, 0)),
            pl.BlockSpec((bsz, n, 3), lambda i: (i, 0, 0)),
            pl.BlockSpec((128, 7), lambda i: (0, 0)),
            pl.BlockSpec((128, 7), lambda i: (0, 0)),
            pl.BlockSpec((128, 5), lambda i: (0, 0)),
            pl.BlockSpec((128, 1), lambda i: (0, 0)),
            pl.BlockSpec((3, 128), lambda i: (0, 0)),
            pl.BlockSpec((3, 7), lambda i: (0, 0)),
            pl.BlockSpec((3, 3), lambda i: (0, 0)),
            pl.BlockSpec((3, 1), lambda i: (0, 0)),
        ],
        out_specs=pl.BlockSpec((bsz, 3, n), lambda i: (i, 0, 0)),
        out_shape=jax.ShapeDtypeStruct((batchsize, 3, n), f32),
        compiler_params=pltpu.CompilerParams(
            dimension_semantics=("parallel",)),
    )(tin, lin, wsrct, wdstt, w5t, bm2, woaggt, woxt, wvt, bo2)
    preds = preds_t.transpose(0, 2, 1)
    return preds, jnp.zeros((batchsize,), dtype=f32)


# speedup vs baseline: 1.1426x; 1.0086x over previous
import functools

import jax
import jax.numpy as jnp
from jax.experimental import pallas as pl
from jax.experimental.pallas import tpu as pltpu

_C0 = 0.28209479177387814
_C1 = 0.4886025119029199
_K = 16


def _nbody_block(t_ref, l_ref, wsrc_ref, wdst_ref, w5_ref, bm_ref,
                 woagg_ref, wox_ref, wv_ref, bo_ref, out_ref,
                 *, bsz, n, wave):
    f32 = jnp.float32
    tblk = t_ref[...]
    massT = tblk[:, 0:1, :]
    locT = tblk[:, 1:4, :]
    velT = tblk[:, 4:7, :]

    wsum = jnp.sum(massT, axis=2, keepdims=True)
    comT = jnp.sum(massT / wsum * locT, axis=2, keepdims=True)
    posT = locT - comT
    vabsT = jnp.sqrt(jnp.sum(velT * velT, axis=1, keepdims=True))

    jsub = jax.lax.broadcasted_iota(jnp.int32, (n, n), 0)
    diag = jsub == jax.lax.broadcasted_iota(jnp.int32, (n, n), 1)
    ones_row = jnp.ones((1, n), dtype=f32)

    for w0 in range(0, bsz, wave):
        _nbody_wave(range(w0, w0 + wave), massT, velT, vabsT, comT, posT,
                    l_ref, wsrc_ref, wdst_ref, w5_ref, bm_ref, woagg_ref,
                    wox_ref, wv_ref, bo_ref, out_ref, jsub, diag, ones_row, n)


def _nbody_wave(sysr, massT, velT, vabsT, comT, posT, l_ref, wsrc_ref,
                wdst_ref, w5_ref, bm_ref, woagg_ref, wox_ref, wv_ref,
                bo_ref, out_ref, jsub, diag, ones_row, n):
    f32 = jnp.float32
    featT, pmT, botT, pw, d2T = {}, {}, {}, {}, {}
    for b in sysr:
        ft = jnp.concatenate(
            [posT[b], velT[b], vabsT[b], massT[b]], axis=0)
        featT[b] = ft
        pmT[b] = jnp.concatenate([posT[b], massT[b]], axis=0)
        botT[b] = jnp.concatenate([ft[0:7], ones_row], axis=0)
        p_tab = jnp.dot(wsrc_ref[...], ft[0:7], preferred_element_type=f32)
        pw[b] = jnp.concatenate(
            [p_tab, w5_ref[...], wdst_ref[...], bm_ref[...]], axis=1)

        pos_s = l_ref[b] - jnp.transpose(comT[b], (1, 0))
        acc = None
        for c in range(3):
            dc = pos_s[:, c:c + 1] - posT[b][c:c + 1, :]
            acc = dc * dc if acc is None else acc + dc * dc
        acc = jnp.where(diag, acc + 1e10, acc)
        ikey = (jax.lax.bitcast_convert_type(acc, jnp.int32)
                & jnp.int32(-128)) | jsub
        d2T[b] = jax.lax.bitcast_convert_type(ikey, f32)

    accm = {b: None for b in sysr}
    accu = {b: None for b in sysr}
    for k in range(_K):
        for b in sysr:
            mn = jnp.min(d2T[b], axis=0, keepdims=True)
            eqT = d2T[b] == mn
            ohf = eqT.astype(f32)
            d2T[b] = d2T[b] + ohf * f32(3.0e38)

            d2e = jax.lax.bitcast_convert_type(
                jax.lax.bitcast_convert_type(mn, jnp.int32)
                & jnp.int32(-128), f32)
            rsq = jax.lax.rsqrt(jnp.maximum(d2e, 1e-16))
            dist = d2e * rsq
            g4 = jnp.dot(pmT[b], ohf, preferred_element_type=f32)
            relT = g4[0:3] - posT[b]
            u = relT * rsq
            mprod = g4[3:4] * massT[b]
            rows = jnp.concatenate([ohf, u, dist, mprod, botT[b]], axis=0)
            pre = jnp.dot(pw[b], rows, preferred_element_type=f32)
            mk = jnp.maximum(pre, 0.0)
            accm[b] = mk if accm[b] is None else accm[b] + mk
            accu[b] = u if accu[b] is None else accu[b] + u

    for b in sysr:
        uvT = featT[b][3:6] / jnp.maximum(featT[b][6:7], 1e-8)
        wvecT = accu[b] * (1.0 / _K) + uvT
        shiftT = (jnp.dot(wox_ref[...], featT[b][0:7],
                          preferred_element_type=f32)
                  + jnp.dot(woagg_ref[...], accm[b],
                            preferred_element_type=f32)
                  + jnp.dot(wv_ref[...], wvecT, preferred_element_type=f32)
                  + bo_ref[...])
        out_ref[b] = posT[b] + shiftT + comT[b]


def kernel(inputs, W_m, b_m, W_o, b_o):
    batchsize, n, _ = inputs.shape
    bsz = 40 if batchsize % 40 == 0 else 1
    f32 = jnp.float32

    tin = inputs.transpose(0, 2, 1)
    lin = inputs[:, :, 1:4]

    wsrct = W_m[0:7].T
    wdstt = W_m[7:14].T
    w5t = jnp.concatenate(
        [_C1 * W_m[jnp.array([17, 15, 16])], W_m[18:20]], axis=0).T
    bm2 = (b_m + _C0 * W_m[14]).reshape(-1, 1)
    woxt = W_o[0:7].T
    woaggt = W_o[7:135].T
    wvt = (_C1 * W_o[jnp.array([138, 136, 137])]).T
    bo2 = (b_o + 2.0 * _C0 * W_o[135]).reshape(-1, 1)

    grid = (batchsize // bsz,)
    body = functools.partial(_nbody_block, bsz=bsz, n=n, wave=bsz)
    preds_t = pl.pallas_call(
        body,
        grid=grid,
        in_specs=[
            pl.BlockSpec((bsz, 7, n), lambda i: (i, 0, 0)),
            pl.BlockSpec((bsz, n, 3), lambda i: (i, 0, 0)),
            pl.BlockSpec((128, 7), lambda i: (0, 0)),
            pl.BlockSpec((128, 7), lambda i: (0, 0)),
            pl.BlockSpec((128, 5), lambda i: (0, 0)),
            pl.BlockSpec((128, 1), lambda i: (0, 0)),
            pl.BlockSpec((3, 128), lambda i: (0, 0)),
            pl.BlockSpec((3, 7), lambda i: (0, 0)),
            pl.BlockSpec((3, 3), lambda i: (0, 0)),
            pl.BlockSpec((3, 1), lambda i: (0, 0)),
        ],
        out_specs=pl.BlockSpec((bsz, 3, n), lambda i: (i, 0, 0)),
        out_shape=jax.ShapeDtypeStruct((batchsize, 3, n), f32),
        compiler_params=pltpu.CompilerParams(
            dimension_semantics=("parallel",)),
    )(tin, lin, wsrct, wdstt, w5t, bm2, woaggt, woxt, wvt, bo2)
    preds = preds_t.transpose(0, 2, 1)
    return preds, jnp.zeros((batchsize,), dtype=f32)
